# baseline (speedup 1.0000x reference)
import functools
import math

import jax
import jax.numpy as jnp
from jax import lax
from jax.experimental import pallas as pl
from jax.experimental.pallas import tpu as pltpu

E = 8
TOP_K = 2
D_MODEL = 1024
HIDDEN = 2048
T = 2048
CAP = 512
NSLOT = CAP - 1
TRASH = E * NSLOT


def _routing_kernel(x_ref, w_ref, b_ref, idx_ref, wts_ref):
    x = x_ref[...]
    w = w_ref[...]
    scores = lax.dot_general(
        w, x, (((0,), (1,)), ((), ())),
        preferred_element_type=jnp.float32,
    ) + b_ref[...].reshape(E, 1)

    eidx = lax.broadcasted_iota(jnp.int32, (E, T), 0)
    v0 = jnp.max(scores, axis=0, keepdims=True)
    i0 = jnp.min(jnp.where(scores == v0, eidx, E), axis=0, keepdims=True)
    masked = jnp.where(eidx == i0, -jnp.inf, scores)
    v1 = jnp.max(masked, axis=0, keepdims=True)
    i1 = jnp.min(jnp.where(masked == v1, eidx, E), axis=0, keepdims=True)
    ed = jnp.exp(v1 - v0)
    denom = 1.0 + ed
    s0 = 1.0 / denom
    s1 = ed / denom

    oh0 = (eidx == i0).astype(jnp.float32)
    oh1 = (eidx == i1).astype(jnp.float32)
    i0 = i0.astype(jnp.float32)
    i1 = i1.astype(jnp.float32)
    c = jnp.concatenate([oh0, oh1], axis=0)
    k = 1
    while k < T:
        shifted = jnp.concatenate(
            [jnp.zeros((2 * E, k), jnp.float32), c[:, : T - k]], axis=1)
        c = c + shifted
        k *= 2
    pos0 = c[:E, :]
    pos1 = pos0 + c[E:, :]
    p0 = jnp.sum(oh0 * pos0, axis=0, keepdims=True)
    p1 = jnp.sum(oh1 * pos1, axis=0, keepdims=True)
    m0 = p0 < float(CAP)
    m1 = p1 < float(CAP)

    disp0 = jnp.where(m0, i0 * float(NSLOT) + p0 - 1.0, float(TRASH))
    disp1 = jnp.where(m1, i1 * float(NSLOT) + p1 - 1.0, float(TRASH))
    comb0 = jnp.where(m0, i0 * float(CAP) + p0, 0.0)
    comb1 = jnp.where(m1, i1 * float(CAP) + p1, 0.0)
    a0 = jnp.where(m0, s0, 0.0)
    a1 = jnp.where(m1, s1, 0.0)
    bb = jnp.where(m0, 0.0, s0) + jnp.where(m1, 0.0, s1)

    idx_ref[...] = jnp.concatenate(
        [disp0, disp1, comb0, comb1], axis=0).astype(jnp.int32)
    wts_ref[...] = jnp.concatenate([a0, a1, bb], axis=0)


def _routing(x2d, router_w, router_b):
    return pl.pallas_call(
        _routing_kernel,
        out_shape=(
            jax.ShapeDtypeStruct((4, T), jnp.int32),
            jax.ShapeDtypeStruct((3, T), jnp.float32),
        ),
    )(x2d, router_w, router_b)


HB = 1024
NHB = HIDDEN // HB


def _ff_kernel(g_ref, w1_ref, w2_ref, w3_ref, out_ref):
    hb = pl.program_id(1)
    g = g_ref[...]
    w1b = w1_ref[0]
    w2b = w2_ref[0]
    w3b = w3_ref[0]
    h = jnp.dot(g, w2b, preferred_element_type=jnp.float32,
                precision=lax.Precision.HIGHEST) * jnp.dot(
        g, w1b, preferred_element_type=jnp.float32,
        precision=lax.Precision.HIGHEST)
    h = jax.nn.gelu(h)
    part = jnp.dot(h, w3b, preferred_element_type=jnp.float32,
                   precision=lax.Precision.HIGHEST)

    @pl.when(hb == 0)
    def _init():
        out_ref[...] = part

    @pl.when(hb != 0)
    def _acc():
        out_ref[...] += part


def _expert_ff(grouped, w1, w2, w3):
    return pl.pallas_call(
        _ff_kernel,
        grid=(E, NHB),
        in_specs=[
            pl.BlockSpec((CAP, D_MODEL), lambda e, h: (e, 0)),
            pl.BlockSpec((1, D_MODEL, HB), lambda e, h: (e, 0, h)),
            pl.BlockSpec((1, D_MODEL, HB), lambda e, h: (e, 0, h)),
            pl.BlockSpec((1, HB, D_MODEL), lambda e, h: (e, h, 0)),
        ],
        out_specs=pl.BlockSpec((CAP, D_MODEL), lambda e, h: (e, 0)),
        out_shape=jax.ShapeDtypeStruct((E * CAP, D_MODEL), jnp.float32),
    )(grouped, w1, w2, w3)


def kernel(x, router_w, router_b, w1, w2, w3):
    x2d = x.reshape(T, D_MODEL)
    idx, wts = _routing(x2d, router_w, router_b)
    disp0, disp1, comb0, comb1 = idx[0], idx[1], idx[2], idx[3]
    a0, a1, bb = wts[0], wts[1], wts[2]

    rows0 = jnp.where(comb0 > 0, comb0, E * CAP)
    rows1 = jnp.where(comb1 > 0, comb1, E * CAP)
    grouped = jnp.zeros((E * CAP + 1, D_MODEL), jnp.float32)
    grouped = grouped.at[rows0].add(x2d).at[rows1].add(x2d)
    grouped = grouped[: E * CAP]

    eo = _expert_ff(grouped, w1, w2, w3)

    out = (a0[:, None] * eo[comb0] + a1[:, None] * eo[comb1]
           + bb[:, None] * x2d)
    return out.reshape(1, T, D_MODEL)


# speedup vs baseline: 2.1671x; 2.1671x over previous
import functools
import math

import jax
import jax.numpy as jnp
from jax import lax
from jax.experimental import pallas as pl
from jax.experimental.pallas import tpu as pltpu

E = 8
TOP_K = 2
D_MODEL = 1024
HIDDEN = 2048
T = 2048
CAP = 512
NSLOT = CAP - 1
TRASH = E * NSLOT


def _routing_kernel(x_ref, w_ref, b_ref, idx_ref, wts_ref):
    x = x_ref[...]
    w = w_ref[...]
    scores = lax.dot_general(
        w, x, (((0,), (1,)), ((), ())),
        preferred_element_type=jnp.float32,
    ) + b_ref[...].reshape(E, 1)

    eidx = lax.broadcasted_iota(jnp.int32, (E, T), 0)
    v0 = jnp.max(scores, axis=0, keepdims=True)
    i0 = jnp.min(jnp.where(scores == v0, eidx, E), axis=0, keepdims=True)
    masked = jnp.where(eidx == i0, -jnp.inf, scores)
    v1 = jnp.max(masked, axis=0, keepdims=True)
    i1 = jnp.min(jnp.where(masked == v1, eidx, E), axis=0, keepdims=True)
    ed = jnp.exp(v1 - v0)
    denom = 1.0 + ed
    s0 = 1.0 / denom
    s1 = ed / denom

    oh0 = (eidx == i0).astype(jnp.float32)
    oh1 = (eidx == i1).astype(jnp.float32)
    i0 = i0.astype(jnp.float32)
    i1 = i1.astype(jnp.float32)
    c = jnp.concatenate([oh0, oh1], axis=0)
    k = 1
    while k < T:
        shifted = jnp.concatenate(
            [jnp.zeros((2 * E, k), jnp.float32), c[:, : T - k]], axis=1)
        c = c + shifted
        k *= 2
    pos0 = c[:E, :]
    pos1 = pos0 + c[E:, :]
    p0 = jnp.sum(oh0 * pos0, axis=0, keepdims=True)
    p1 = jnp.sum(oh1 * pos1, axis=0, keepdims=True)
    m0 = p0 < float(CAP)
    m1 = p1 < float(CAP)

    disp0 = jnp.where(m0, i0 * float(NSLOT) + p0 - 1.0, float(TRASH))
    disp1 = jnp.where(m1, i1 * float(NSLOT) + p1 - 1.0, float(TRASH))
    comb0 = jnp.where(m0, i0 * float(CAP) + p0, 0.0)
    comb1 = jnp.where(m1, i1 * float(CAP) + p1, 0.0)
    a0 = jnp.where(m0, s0, 0.0)
    a1 = jnp.where(m1, s1, 0.0)
    bb = jnp.where(m0, 0.0, s0) + jnp.where(m1, 0.0, s1)

    idx_ref[...] = jnp.concatenate(
        [disp0, disp1, comb0, comb1], axis=0).astype(jnp.int32)
    wts_ref[...] = jnp.concatenate([a0, a1, bb], axis=0)


def _routing(x2d, router_w, router_b):
    return pl.pallas_call(
        _routing_kernel,
        out_shape=(
            jax.ShapeDtypeStruct((4, T), jnp.int32),
            jax.ShapeDtypeStruct((3, T), jnp.float32),
        ),
    )(x2d, router_w, router_b)


HB = 1024
NHB = HIDDEN // HB


def _ff_kernel(g_ref, w1_ref, w2_ref, w3_ref, out_ref):
    hb = pl.program_id(1)
    g = g_ref[...].astype(jnp.bfloat16)
    w1b = w1_ref[0].astype(jnp.bfloat16)
    w2b = w2_ref[0].astype(jnp.bfloat16)
    w3b = w3_ref[0].astype(jnp.bfloat16)
    h = jnp.dot(g, w2b, preferred_element_type=jnp.float32) * jnp.dot(
        g, w1b, preferred_element_type=jnp.float32)
    h = jax.nn.gelu(h).astype(jnp.bfloat16)
    part = jnp.dot(h, w3b, preferred_element_type=jnp.float32)

    @pl.when(hb == 0)
    def _init():
        out_ref[...] = part

    @pl.when(hb != 0)
    def _acc():
        out_ref[...] += part


def _expert_ff(grouped, w1, w2, w3):
    return pl.pallas_call(
        _ff_kernel,
        grid=(E, NHB),
        in_specs=[
            pl.BlockSpec((CAP, D_MODEL), lambda e, h: (e, 0)),
            pl.BlockSpec((1, D_MODEL, HB), lambda e, h: (e, 0, h)),
            pl.BlockSpec((1, D_MODEL, HB), lambda e, h: (e, 0, h)),
            pl.BlockSpec((1, HB, D_MODEL), lambda e, h: (e, h, 0)),
        ],
        out_specs=pl.BlockSpec((CAP, D_MODEL), lambda e, h: (e, 0)),
        out_shape=jax.ShapeDtypeStruct((E * CAP, D_MODEL), jnp.float32),
    )(grouped, w1, w2, w3)


def kernel(x, router_w, router_b, w1, w2, w3):
    x2d = x.reshape(T, D_MODEL)
    idx, wts = _routing(x2d, router_w, router_b)
    disp0, disp1, comb0, comb1 = idx[0], idx[1], idx[2], idx[3]
    a0, a1, bb = wts[0], wts[1], wts[2]

    rows0 = jnp.where(comb0 > 0, comb0, E * CAP)
    rows1 = jnp.where(comb1 > 0, comb1, E * CAP)
    grouped = jnp.zeros((E * CAP + 1, D_MODEL), jnp.float32)
    grouped = grouped.at[rows0].add(x2d).at[rows1].add(x2d)
    grouped = grouped[: E * CAP]

    eo = _expert_ff(grouped, w1, w2, w3)

    out = (a0[:, None] * eo[comb0] + a1[:, None] * eo[comb1]
           + bb[:, None] * x2d)
    return out.reshape(1, T, D_MODEL)
